# baseline (speedup 1.0000x reference)
import functools

import jax
import jax.numpy as jnp
from jax import lax
from jax.experimental import pallas as pl
from jax.experimental.pallas import tpu as pltpu
from jax.experimental.pallas import tpu_sc as plsc

N = 10000
E = 160000
D_NODE = 64
D_EDGE = 16
H = 32
G = 64

NC = 2
NS = 16
NW = NC * NS
ROWS_DMA = 128
E_PAD = 163840
PER_W = E_PAD // NW
K_DMA = PER_W // ROWS_DMA
GROUP = 1024
N_GROUPS = PER_W // GROUP
DMA_PER_GROUP = GROUP // ROWS_DMA
NP = 10240
ROWS_PER_TILE = NP // NS

@functools.lru_cache(maxsize=None)
def _sc_mesh():
    return plsc.VectorSubcoreMesh(
        core_axis_name="c", subcore_axis_name="s",
        num_cores=NC, num_subcores=NS)


@functools.lru_cache(maxsize=None)
def _make_gather(d_in):
    group_rows = 32768 // d_in
    dpg = group_rows // ROWS_DMA
    ngr = PER_W // group_rows

    def body(table_hbm, src_hbm, out_hbm, idx_v, rows_v, semg, semo):
        wid = lax.axis_index("s") * NC + lax.axis_index("c")
        base = wid * PER_W
        pltpu.sync_copy(src_hbm.at[wid], idx_v)

        def fire(g, b):
            for j in range(dpg):
                pltpu.async_copy(
                    table_hbm.at[idx_v.at[g * dpg + j]],
                    rows_v.at[b, pl.ds(j * ROWS_DMA, ROWS_DMA)], semg)

        fire(0, 0)

        def group(g, carry):
            b = lax.rem(g, 2)
            nb = lax.rem(g + 1, 2)
            for j in range(dpg):
                pltpu.make_async_copy(
                    table_hbm.at[idx_v.at[j]],
                    rows_v.at[b, pl.ds(j * ROWS_DMA, ROWS_DMA)], semg).wait()

            @pl.when(g >= 1)
            def _():
                pltpu.make_async_copy(
                    rows_v.at[nb], out_hbm.at[pl.ds(base, group_rows)],
                    semo).wait()

            @pl.when(g + 1 < ngr)
            def _():
                fire(g + 1, nb)

            pltpu.async_copy(
                rows_v.at[b],
                out_hbm.at[pl.ds(base + g * group_rows, group_rows)], semo)
            return carry

        lax.fori_loop(0, ngr, group, 0)
        pltpu.make_async_copy(
            rows_v.at[lax.rem(ngr - 1, 2)],
            out_hbm.at[pl.ds(base, group_rows)], semo).wait()

    return pl.kernel(
        body,
        out_type=jax.ShapeDtypeStruct((E_PAD, d_in), jnp.float32),
        mesh=_sc_mesh(),
        compiler_params=pltpu.CompilerParams(use_tc_tiling_on_sc=False),
        scratch_types=[
            pltpu.VMEM((K_DMA, ROWS_DMA), jnp.int32),
            pltpu.VMEM((2, group_rows, d_in), jnp.float32),
            pltpu.SemaphoreType.DMA,
            pltpu.SemaphoreType.DMA,
        ],
    )


def _scatter_body(msg_hbm, dst_hbm, zeros_hbm, out_hbm,
                  idx_v, msg_v, acc_sh, seml, sems):
    cid = lax.axis_index("c")
    sid = lax.axis_index("s")
    wid = sid * NC + cid
    base = wid * PER_W
    row0 = sid * ROWS_PER_TILE
    pltpu.sync_copy(dst_hbm.at[wid], idx_v)
    pltpu.async_copy(msg_hbm.at[pl.ds(base, GROUP)], msg_v.at[0], seml)
    pltpu.sync_copy(zeros_hbm, acc_sh.at[pl.ds(row0, ROWS_PER_TILE)])
    plsc.subcore_barrier()

    def fire_scatters(g, b):
        for j in range(DMA_PER_GROUP):
            pltpu.async_copy(msg_v.at[b, pl.ds(j * ROWS_DMA, ROWS_DMA)],
                             acc_sh.at[idx_v.at[g * DMA_PER_GROUP + j]],
                             sems, add=True)

    def drain_scatters(b):
        for j in range(DMA_PER_GROUP):
            pltpu.make_async_copy(
                msg_v.at[b, pl.ds(j * ROWS_DMA, ROWS_DMA)],
                acc_sh.at[idx_v.at[j]], sems).wait()

    def group(g, carry):
        b = lax.rem(g, 2)
        nb = lax.rem(g + 1, 2)
        pltpu.make_async_copy(
            msg_hbm.at[pl.ds(base, GROUP)], msg_v.at[b], seml).wait()

        @pl.when(g >= 1)
        def _():
            drain_scatters(nb)

        @pl.when(g + 1 < N_GROUPS)
        def _():
            pltpu.async_copy(
                msg_hbm.at[pl.ds(base + (g + 1) * GROUP, GROUP)],
                msg_v.at[nb], seml)

        fire_scatters(g, b)
        return carry

    lax.fori_loop(0, N_GROUPS, group, 0)
    drain_scatters(lax.rem(N_GROUPS - 1, 2))
    plsc.subcore_barrier()
    pltpu.sync_copy(acc_sh.at[pl.ds(row0, ROWS_PER_TILE)],
                    out_hbm.at[pl.ds(cid * NP + row0, ROWS_PER_TILE)])


@functools.lru_cache(maxsize=None)
def _make_scatter():
    return pl.kernel(
        _scatter_body,
        out_type=jax.ShapeDtypeStruct((NC * NP, H), jnp.float32),
        mesh=_sc_mesh(),
        compiler_params=pltpu.CompilerParams(use_tc_tiling_on_sc=False),
        scratch_types=[
            pltpu.VMEM((K_DMA, ROWS_DMA), jnp.int32),
            pltpu.VMEM((2, GROUP, H), jnp.float32),
            pltpu.VMEM_SHARED((NP, H), jnp.float32),
            pltpu.SemaphoreType.DMA,
            pltpu.SemaphoreType.DMA,
        ],
    )


BE = 2048


NK = D_EDGE + 1
KW = NK * H


def _msg_body(xj_ref, ea_ref, w_ref, r_ref, out_ref):
    y = jnp.dot(xj_ref[...], w_ref[...],
                preferred_element_type=jnp.float32)
    c = jnp.dot(ea_ref[...], r_ref[...],
                preferred_element_type=jnp.float32)
    t = c * y
    a = t[:, :256] + t[:, 256:512]
    a = a[:, :128] + a[:, 128:]
    a = a[:, :64] + a[:, 64:]
    out_ref[...] = a[:, :H] + a[:, H:] + t[:, 512:544]


def _make_msg(d_in):
    return pl.pallas_call(
        _msg_body,
        grid=(E_PAD // BE,),
        in_specs=[
            pl.BlockSpec((BE, d_in), lambda i: (i, 0)),
            pl.BlockSpec((BE, NK), lambda i: (i, 0)),
            pl.BlockSpec((d_in, KW), lambda i: (0, 0)),
            pl.BlockSpec((NK, KW), lambda i: (0, 0)),
        ],
        out_specs=pl.BlockSpec((BE, H), lambda i: (i, 0)),
        out_shape=jax.ShapeDtypeStruct((E_PAD, H), jnp.float32),
    )


_msg64 = _make_msg(D_NODE)
_msg32 = _make_msg(H)


def _update_body(x_ref, agg_ref, root_ref, bias_ref, out_ref):
    agg = agg_ref[pl.ds(0, N)] + agg_ref[pl.ds(NP, N)]
    h = (jnp.dot(x_ref[...], root_ref[...], preferred_element_type=jnp.float32)
         + agg + bias_ref[...])
    out_ref[...] = jnp.maximum(h, 0.0)


def _make_update(d_in):
    return pl.pallas_call(
        _update_body,
        out_shape=jax.ShapeDtypeStruct((N, H), jnp.float32),
    )


_update1 = _make_update(D_NODE)


def _final_body(h1_ref, agg_ref, root_ref, bias_ref, batch_ref,
                l1w_ref, l1b_ref, l2w_ref, l2b_ref, out_ref):
    agg = agg_ref[pl.ds(0, N)] + agg_ref[pl.ds(NP, N)]
    h2 = (jnp.dot(h1_ref[...], root_ref[...],
                  preferred_element_type=jnp.float32)
          + agg + bias_ref[...])
    gids = jnp.broadcast_to(batch_ref[...], (G, N))
    oht = (lax.broadcasted_iota(jnp.int32, (G, N), 0) == gids
           ).astype(jnp.float32)
    sums = jnp.dot(oht, h2, preferred_element_type=jnp.float32)
    cnt = jnp.sum(oht, axis=1, keepdims=True)
    pooled = sums / jnp.maximum(cnt, 1.0)
    z = jnp.maximum(
        jnp.dot(pooled, l1w_ref[...], preferred_element_type=jnp.float32)
        + l1b_ref[...], 0.0)
    out_ref[...] = jax.nn.sigmoid(
        jnp.dot(z, l2w_ref[...], preferred_element_type=jnp.float32)
        + l2b_ref[...])


_final = pl.pallas_call(
    _final_body,
    out_shape=jax.ShapeDtypeStruct((G, 1), jnp.float32),
)


def kernel(x, edge_index, edge_attr, batch,
           nn1_W, nn1_b, root1, bias1,
           nn2_W, nn2_b, root2, bias2,
           lin1_W, lin1_b, lin2_W, lin2_b):
    src = edge_index[0].astype(jnp.int32)
    dst = edge_index[1].astype(jnp.int32)
    pad = E_PAD - E
    src_p = jnp.concatenate([src, jnp.zeros((pad,), jnp.int32)]
                            ).reshape(NW, K_DMA, ROWS_DMA)
    dst_p = jnp.concatenate([dst, jnp.full((pad,), NP - 1, jnp.int32)]
                            ).reshape(NW, K_DMA, ROWS_DMA)
    ea1 = jnp.concatenate(
        [edge_attr, jnp.ones((E, 1), jnp.float32)], axis=1)
    ea1 = jnp.concatenate([ea1, jnp.zeros((pad, NK), jnp.float32)], axis=0)

    def wcat(nnW, nnb, d_in):
        w = nnW.reshape(D_EDGE, d_in, H).transpose(1, 0, 2).reshape(d_in, -1)
        w = jnp.concatenate([w, nnb.reshape(d_in, H)], axis=1)
        return jnp.pad(w, ((0, 0), (0, KW - w.shape[1])))

    wall1 = wcat(nn1_W, nn1_b, D_NODE)
    wall2 = wcat(nn2_W, nn2_b, H)
    rmat = jnp.repeat(jnp.eye(NK, dtype=jnp.float32), H, axis=1)
    zeros_np = jnp.zeros((ROWS_PER_TILE, H), jnp.float32)
    batch2d = batch.astype(jnp.int32).reshape(1, N)

    xj = _make_gather(D_NODE)(x, src_p)
    msg1 = _msg64(xj, ea1, wall1, rmat)
    agg1 = _make_scatter()(msg1, dst_p, zeros_np)
    h1 = _update1(x, agg1, root1, bias1.reshape(1, H))

    hj = _make_gather(H)(h1, src_p)
    msg2 = _msg32(hj, ea1, wall2, rmat)
    agg2 = _make_scatter()(msg2, dst_p, zeros_np)

    return _final(h1, agg2, root2, bias2.reshape(1, H),
                  batch2d, lin1_W, lin1_b.reshape(1, H // 2),
                  lin2_W, lin2_b.reshape(1, 1))


# speedup vs baseline: 1.2356x; 1.2356x over previous
import functools

import jax
import jax.numpy as jnp
from jax import lax
from jax.experimental import pallas as pl
from jax.experimental.pallas import tpu as pltpu
from jax.experimental.pallas import tpu_sc as plsc

N = 10000
E = 160000
D_NODE = 64
D_EDGE = 16
H = 32
G = 64

NC = 2
NS = 16
NW = NC * NS
ROWS_DMA = 128
E_PAD = 163840
PER_W = E_PAD // NW
K_DMA = PER_W // ROWS_DMA
GROUP = 1024
N_GROUPS = PER_W // GROUP
DMA_PER_GROUP = GROUP // ROWS_DMA
NP = 10240
ROWS_PER_TILE = NP // NS

@functools.lru_cache(maxsize=None)
def _sc_mesh():
    return plsc.VectorSubcoreMesh(
        core_axis_name="c", subcore_axis_name="s",
        num_cores=NC, num_subcores=NS)


@functools.lru_cache(maxsize=None)
def _make_gather(d_in):
    group_rows = 32768 // d_in
    dpg = group_rows // ROWS_DMA
    ngr = PER_W // group_rows
    rpt = 632
    last = N - rpt * (NS - 1)

    def body(table_hbm, src_hbm, out_hbm, idx_v, rows_v, tab_sh, semg, semo):
        sid = lax.axis_index("s")
        wid = sid * NC + lax.axis_index("c")
        base = wid * PER_W
        pltpu.sync_copy(src_hbm.at[wid], idx_v)

        @pl.when(sid < NS - 1)
        def _():
            pltpu.sync_copy(table_hbm.at[pl.ds(sid * rpt, rpt)],
                            tab_sh.at[pl.ds(sid * rpt, rpt)])

        @pl.when(sid == NS - 1)
        def _():
            pltpu.sync_copy(table_hbm.at[pl.ds((NS - 1) * rpt, last)],
                            tab_sh.at[pl.ds((NS - 1) * rpt, last)])

        plsc.subcore_barrier()

        def fire(g, b):
            for j in range(dpg):
                pltpu.async_copy(
                    tab_sh.at[idx_v.at[g * dpg + j]],
                    rows_v.at[b, pl.ds(j * ROWS_DMA, ROWS_DMA)], semg)

        fire(0, 0)

        def group(g, carry):
            b = lax.rem(g, 2)
            nb = lax.rem(g + 1, 2)
            for j in range(dpg):
                pltpu.make_async_copy(
                    tab_sh.at[idx_v.at[j]],
                    rows_v.at[b, pl.ds(j * ROWS_DMA, ROWS_DMA)], semg).wait()

            @pl.when(g >= 1)
            def _():
                pltpu.make_async_copy(
                    rows_v.at[nb], out_hbm.at[pl.ds(base, group_rows)],
                    semo).wait()

            @pl.when(g + 1 < ngr)
            def _():
                fire(g + 1, nb)

            pltpu.async_copy(
                rows_v.at[b],
                out_hbm.at[pl.ds(base + g * group_rows, group_rows)], semo)
            return carry

        lax.fori_loop(0, ngr, group, 0)
        pltpu.make_async_copy(
            rows_v.at[lax.rem(ngr - 1, 2)],
            out_hbm.at[pl.ds(base, group_rows)], semo).wait()

    return pl.kernel(
        body,
        out_type=jax.ShapeDtypeStruct((E_PAD, d_in), jnp.float32),
        mesh=_sc_mesh(),
        compiler_params=pltpu.CompilerParams(use_tc_tiling_on_sc=False),
        scratch_types=[
            pltpu.VMEM((K_DMA, ROWS_DMA), jnp.int32),
            pltpu.VMEM((2, group_rows, d_in), jnp.float32),
            pltpu.VMEM_SHARED((N, d_in), jnp.float32),
            pltpu.SemaphoreType.DMA,
            pltpu.SemaphoreType.DMA,
        ],
    )


def _scatter_body(msg_hbm, dst_hbm, zeros_hbm, out_hbm,
                  idx_v, msg_v, acc_sh, seml, sems):
    cid = lax.axis_index("c")
    sid = lax.axis_index("s")
    wid = sid * NC + cid
    base = wid * PER_W
    row0 = sid * ROWS_PER_TILE
    pltpu.sync_copy(dst_hbm.at[wid], idx_v)
    pltpu.async_copy(msg_hbm.at[pl.ds(base, GROUP)], msg_v.at[0], seml)
    pltpu.sync_copy(zeros_hbm, acc_sh.at[pl.ds(row0, ROWS_PER_TILE)])
    plsc.subcore_barrier()

    def fire_scatters(g, b):
        for j in range(DMA_PER_GROUP):
            pltpu.async_copy(msg_v.at[b, pl.ds(j * ROWS_DMA, ROWS_DMA)],
                             acc_sh.at[idx_v.at[g * DMA_PER_GROUP + j]],
                             sems, add=True)

    def drain_scatters(b):
        for j in range(DMA_PER_GROUP):
            pltpu.make_async_copy(
                msg_v.at[b, pl.ds(j * ROWS_DMA, ROWS_DMA)],
                acc_sh.at[idx_v.at[j]], sems).wait()

    def group(g, carry):
        b = lax.rem(g, 2)
        nb = lax.rem(g + 1, 2)
        pltpu.make_async_copy(
            msg_hbm.at[pl.ds(base, GROUP)], msg_v.at[b], seml).wait()

        @pl.when(g >= 1)
        def _():
            drain_scatters(nb)

        @pl.when(g + 1 < N_GROUPS)
        def _():
            pltpu.async_copy(
                msg_hbm.at[pl.ds(base + (g + 1) * GROUP, GROUP)],
                msg_v.at[nb], seml)

        fire_scatters(g, b)
        return carry

    lax.fori_loop(0, N_GROUPS, group, 0)
    drain_scatters(lax.rem(N_GROUPS - 1, 2))
    plsc.subcore_barrier()
    pltpu.sync_copy(acc_sh.at[pl.ds(row0, ROWS_PER_TILE)],
                    out_hbm.at[pl.ds(cid * NP + row0, ROWS_PER_TILE)])


@functools.lru_cache(maxsize=None)
def _make_scatter():
    return pl.kernel(
        _scatter_body,
        out_type=jax.ShapeDtypeStruct((NC * NP, H), jnp.float32),
        mesh=_sc_mesh(),
        compiler_params=pltpu.CompilerParams(use_tc_tiling_on_sc=False),
        scratch_types=[
            pltpu.VMEM((K_DMA, ROWS_DMA), jnp.int32),
            pltpu.VMEM((2, GROUP, H), jnp.float32),
            pltpu.VMEM_SHARED((NP, H), jnp.float32),
            pltpu.SemaphoreType.DMA,
            pltpu.SemaphoreType.DMA,
        ],
    )


BE = 2048


NK = D_EDGE + 1
KW = NK * H
GRID_E = -(-E // BE)


def _msg_body(xj_ref, ea_ref, w_ref, r_ref, out_ref):
    y = jnp.dot(xj_ref[...], w_ref[...],
                preferred_element_type=jnp.float32)
    c = jnp.dot(ea_ref[...], r_ref[...],
                preferred_element_type=jnp.float32)
    t = c * y[:, :D_EDGE * H]
    a = t[:, :256] + t[:, 256:512]
    a = a[:, :128] + a[:, 128:]
    a = a[:, :64] + a[:, 64:]
    out_ref[...] = a[:, :H] + a[:, H:] + y[:, D_EDGE * H:]


def _make_msg(d_in):
    return pl.pallas_call(
        _msg_body,
        grid=(GRID_E,),
        in_specs=[
            pl.BlockSpec((BE, d_in), lambda i: (i, 0)),
            pl.BlockSpec((BE, D_EDGE), lambda i: (i, 0)),
            pl.BlockSpec((d_in, KW), lambda i: (0, 0)),
            pl.BlockSpec((D_EDGE, D_EDGE * H), lambda i: (0, 0)),
        ],
        out_specs=pl.BlockSpec((BE, H), lambda i: (i, 0)),
        out_shape=jax.ShapeDtypeStruct((E_PAD, H), jnp.float32),
    )


_msg64 = _make_msg(D_NODE)
_msg32 = _make_msg(H)


def _update_body(x_ref, agg_ref, root_ref, bias_ref, out_ref):
    agg = agg_ref[pl.ds(0, N)] + agg_ref[pl.ds(NP, N)]
    h = (jnp.dot(x_ref[...], root_ref[...], preferred_element_type=jnp.float32)
         + agg + bias_ref[...])
    out_ref[...] = jnp.maximum(h, 0.0)


def _make_update(d_in):
    return pl.pallas_call(
        _update_body,
        out_shape=jax.ShapeDtypeStruct((N, H), jnp.float32),
    )


_update1 = _make_update(D_NODE)


def _final_body(h1_ref, agg_ref, root_ref, bias_ref, batch_ref,
                l1w_ref, l1b_ref, l2w_ref, l2b_ref, out_ref):
    agg = agg_ref[pl.ds(0, N)] + agg_ref[pl.ds(NP, N)]
    h2 = (jnp.dot(h1_ref[...], root_ref[...],
                  preferred_element_type=jnp.float32)
          + agg + bias_ref[...])
    gids = jnp.broadcast_to(batch_ref[...], (G, N))
    oht = (lax.broadcasted_iota(jnp.int32, (G, N), 0) == gids
           ).astype(jnp.float32)
    sums = jnp.dot(oht, h2, preferred_element_type=jnp.float32)
    cnt = jnp.sum(oht, axis=1, keepdims=True)
    pooled = sums / jnp.maximum(cnt, 1.0)
    z = jnp.maximum(
        jnp.dot(pooled, l1w_ref[...], preferred_element_type=jnp.float32)
        + l1b_ref[...], 0.0)
    out_ref[...] = jax.nn.sigmoid(
        jnp.dot(z, l2w_ref[...], preferred_element_type=jnp.float32)
        + l2b_ref[...])


_final = pl.pallas_call(
    _final_body,
    out_shape=jax.ShapeDtypeStruct((G, 1), jnp.float32),
)


def kernel(x, edge_index, edge_attr, batch,
           nn1_W, nn1_b, root1, bias1,
           nn2_W, nn2_b, root2, bias2,
           lin1_W, lin1_b, lin2_W, lin2_b):
    src = edge_index[0].astype(jnp.int32)
    dst = edge_index[1].astype(jnp.int32)
    pad = E_PAD - E
    src_p = jnp.concatenate([src, jnp.zeros((pad,), jnp.int32)]
                            ).reshape(NW, K_DMA, ROWS_DMA)
    dst_p = jnp.concatenate([dst, jnp.full((pad,), NP - 1, jnp.int32)]
                            ).reshape(NW, K_DMA, ROWS_DMA)
    def wcat(nnW, nnb, d_in):
        w = nnW.reshape(D_EDGE, d_in, H).transpose(1, 0, 2).reshape(d_in, -1)
        return jnp.concatenate([w, nnb.reshape(d_in, H)], axis=1)

    wall1 = wcat(nn1_W, nn1_b, D_NODE)
    wall2 = wcat(nn2_W, nn2_b, H)
    rmat = jnp.repeat(jnp.eye(D_EDGE, dtype=jnp.float32), H, axis=1)
    zeros_np = jnp.zeros((ROWS_PER_TILE, H), jnp.float32)
    batch2d = batch.astype(jnp.int32).reshape(1, N)

    xj = _make_gather(D_NODE)(x, src_p)
    msg1 = _msg64(xj, edge_attr, wall1, rmat)
    agg1 = _make_scatter()(msg1, dst_p, zeros_np)
    h1 = _update1(x, agg1, root1, bias1.reshape(1, H))

    hj = _make_gather(H)(h1, src_p)
    msg2 = _msg32(hj, edge_attr, wall2, rmat)
    agg2 = _make_scatter()(msg2, dst_p, zeros_np)

    return _final(h1, agg2, root2, bias2.reshape(1, H),
                  batch2d, lin1_W, lin1_b.reshape(1, H // 2),
                  lin2_W, lin2_b.reshape(1, 1))


# speedup vs baseline: 1.6919x; 1.3692x over previous
import functools

import jax
import jax.numpy as jnp
from jax import lax
from jax.experimental import pallas as pl
from jax.experimental.pallas import tpu as pltpu
from jax.experimental.pallas import tpu_sc as plsc

N = 10000
E = 160000
D_NODE = 64
D_EDGE = 16
H = 32
G = 64

NC = 2
NS = 16
NW = NC * NS
ROWS_DMA = 128
E_PAD = 163840
PER_W = E_PAD // NW
K_DMA = PER_W // ROWS_DMA
GROUP = 1024
N_GROUPS = PER_W // GROUP
DMA_PER_GROUP = GROUP // ROWS_DMA
NP = 10240
ROWS_PER_TILE = NP // NS

@functools.lru_cache(maxsize=None)
def _sc_mesh():
    return plsc.VectorSubcoreMesh(
        core_axis_name="c", subcore_axis_name="s",
        num_cores=NC, num_subcores=NS)


@functools.lru_cache(maxsize=None)
def _make_gather(d_in):
    group_rows = 32768 // d_in
    dpg = group_rows // ROWS_DMA
    ngr = PER_W // group_rows
    rpt = 632
    last = N - rpt * (NS - 1)

    def body(table_hbm, src_hbm, out_hbm, idx_v, rows_v, tab_sh, semg, semo):
        sid = lax.axis_index("s")
        wid = sid * NC + lax.axis_index("c")
        base = wid * PER_W
        pltpu.sync_copy(src_hbm.at[wid], idx_v)

        @pl.when(sid < NS - 1)
        def _():
            pltpu.sync_copy(table_hbm.at[pl.ds(sid * rpt, rpt)],
                            tab_sh.at[pl.ds(sid * rpt, rpt)])

        @pl.when(sid == NS - 1)
        def _():
            pltpu.sync_copy(table_hbm.at[pl.ds((NS - 1) * rpt, last)],
                            tab_sh.at[pl.ds((NS - 1) * rpt, last)])

        plsc.subcore_barrier()

        def fire(g, b):
            for j in range(dpg):
                pltpu.async_copy(
                    tab_sh.at[idx_v.at[g * dpg + j]],
                    rows_v.at[b, pl.ds(j * ROWS_DMA, ROWS_DMA)], semg)

        fire(0, 0)

        def group(g, carry):
            b = lax.rem(g, 2)
            nb = lax.rem(g + 1, 2)
            for j in range(dpg):
                pltpu.make_async_copy(
                    tab_sh.at[idx_v.at[j]],
                    rows_v.at[b, pl.ds(j * ROWS_DMA, ROWS_DMA)], semg).wait()

            @pl.when(g >= 1)
            def _():
                pltpu.make_async_copy(
                    rows_v.at[nb], out_hbm.at[pl.ds(base, group_rows)],
                    semo).wait()

            @pl.when(g + 1 < ngr)
            def _():
                fire(g + 1, nb)

            pltpu.async_copy(
                rows_v.at[b],
                out_hbm.at[pl.ds(base + g * group_rows, group_rows)], semo)
            return carry

        lax.fori_loop(0, ngr, group, 0)
        pltpu.make_async_copy(
            rows_v.at[lax.rem(ngr - 1, 2)],
            out_hbm.at[pl.ds(base, group_rows)], semo).wait()

    return pl.kernel(
        body,
        out_type=jax.ShapeDtypeStruct((E_PAD, d_in), jnp.float32),
        mesh=_sc_mesh(),
        compiler_params=pltpu.CompilerParams(use_tc_tiling_on_sc=False),
        scratch_types=[
            pltpu.VMEM((K_DMA, ROWS_DMA), jnp.int32),
            pltpu.VMEM((2, group_rows, d_in), jnp.float32),
            pltpu.VMEM_SHARED((N, d_in), jnp.float32),
            pltpu.SemaphoreType.DMA,
            pltpu.SemaphoreType.DMA,
        ],
    )


def _scatter_body(msg_hbm, dst_hbm, zeros_hbm, out_hbm,
                  idx_v, msg_v, acc_sh, seml, sems):
    cid = lax.axis_index("c")
    sid = lax.axis_index("s")
    wid = sid * NC + cid
    base = wid * PER_W
    row0 = sid * ROWS_PER_TILE
    pltpu.sync_copy(dst_hbm.at[wid], idx_v)
    pltpu.async_copy(msg_hbm.at[pl.ds(base, GROUP)], msg_v.at[0], seml)
    pltpu.sync_copy(zeros_hbm, acc_sh.at[pl.ds(row0, ROWS_PER_TILE)])
    plsc.subcore_barrier()

    def fire_scatters(g, b):
        for j in range(DMA_PER_GROUP):
            pltpu.async_copy(msg_v.at[b, pl.ds(j * ROWS_DMA, ROWS_DMA)],
                             acc_sh.at[idx_v.at[g * DMA_PER_GROUP + j]],
                             sems, add=True)

    def drain_scatters(b):
        for j in range(DMA_PER_GROUP):
            pltpu.make_async_copy(
                msg_v.at[b, pl.ds(j * ROWS_DMA, ROWS_DMA)],
                acc_sh.at[idx_v.at[j]], sems).wait()

    def group(g, carry):
        b = lax.rem(g, 2)
        nb = lax.rem(g + 1, 2)
        pltpu.make_async_copy(
            msg_hbm.at[pl.ds(base, GROUP)], msg_v.at[b], seml).wait()

        @pl.when(g >= 1)
        def _():
            drain_scatters(nb)

        @pl.when(g + 1 < N_GROUPS)
        def _():
            pltpu.async_copy(
                msg_hbm.at[pl.ds(base + (g + 1) * GROUP, GROUP)],
                msg_v.at[nb], seml)

        fire_scatters(g, b)
        return carry

    lax.fori_loop(0, N_GROUPS, group, 0)
    drain_scatters(lax.rem(N_GROUPS - 1, 2))
    plsc.subcore_barrier()
    pltpu.sync_copy(acc_sh.at[pl.ds(row0, ROWS_PER_TILE)],
                    out_hbm.at[pl.ds(cid * NP + row0, ROWS_PER_TILE)])


@functools.lru_cache(maxsize=None)
def _make_scatter():
    return pl.kernel(
        _scatter_body,
        out_type=jax.ShapeDtypeStruct((NC * NP, H), jnp.float32),
        mesh=_sc_mesh(),
        compiler_params=pltpu.CompilerParams(use_tc_tiling_on_sc=False),
        scratch_types=[
            pltpu.VMEM((K_DMA, ROWS_DMA), jnp.int32),
            pltpu.VMEM((2, GROUP, H), jnp.float32),
            pltpu.VMEM_SHARED((NP, H), jnp.float32),
            pltpu.SemaphoreType.DMA,
            pltpu.SemaphoreType.DMA,
        ],
    )


BE = 2048


KW = (D_EDGE + 1) * H
BE4 = 512
Q4 = E_PAD // 4
GRID_M = Q4 // BE4
NBLK = GRID_M


def _fold(c, y):
    t = c * y[:, :D_EDGE * H]
    a = t[:, :256] + t[:, 256:512]
    a = a[:, :128] + a[:, 128:]
    a = a[:, :64] + a[:, 64:]
    return a[:, :H] + a[:, H:] + y[:, D_EDGE * H:]


def _cexp(eat, r_ref):
    return jax.lax.dot_general(eat, r_ref[...], (((0,), (0,)), ((), ())),
                               preferred_element_type=jnp.float32)


def _msg64_body(xa_ref, xb_ref, ea0_ref, ea1_ref, ea2_ref, ea3_ref,
                w_ref, r_ref, out_ref):
    blka = xa_ref[...]
    blkb = xb_ref[...]
    wl = w_ref[pl.ds(0, 128), :]
    wr = w_ref[pl.ds(128, 128), :]
    m0 = _fold(_cexp(ea0_ref[...], r_ref),
               jnp.dot(blka, wl, preferred_element_type=jnp.float32))
    m1 = _fold(_cexp(ea1_ref[...], r_ref),
               jnp.dot(blkb, wl, preferred_element_type=jnp.float32))
    m2 = _fold(_cexp(ea2_ref[...], r_ref),
               jnp.dot(blka, wr, preferred_element_type=jnp.float32))
    m3 = _fold(_cexp(ea3_ref[...], r_ref),
               jnp.dot(blkb, wr, preferred_element_type=jnp.float32))
    out_ref[...] = jnp.concatenate([m0, m1, m2, m3], axis=1)


def _msg32_body(xj_ref, ea0_ref, ea1_ref, ea2_ref, ea3_ref,
                w_ref, r_ref, out_ref):
    blk = xj_ref[...]
    ea_refs = (ea0_ref, ea1_ref, ea2_ref, ea3_ref)
    msgs = []
    for q in range(4):
        y = jnp.dot(blk, w_ref[pl.ds(q * 128, 128), :],
                    preferred_element_type=jnp.float32)
        msgs.append(_fold(_cexp(ea_refs[q][...], r_ref), y))
    out_ref[...] = jnp.concatenate(msgs, axis=1)


def _ea_specs():
    return [pl.BlockSpec((D_EDGE, BE4), lambda i, q=q: (0, q * NBLK + i))
            for q in range(4)]


_msg64 = pl.pallas_call(
    _msg64_body,
    grid=(GRID_M,),
    in_specs=[
        pl.BlockSpec((BE4, 128), lambda i: (i, 0)),
        pl.BlockSpec((BE4, 128), lambda i: (GRID_M + i, 0)),
        *_ea_specs(),
        pl.BlockSpec((256, KW), lambda i: (0, 0)),
        pl.BlockSpec((D_EDGE, D_EDGE * H), lambda i: (0, 0)),
    ],
    out_specs=pl.BlockSpec((BE4, 4 * H), lambda i: (i, 0)),
    out_shape=jax.ShapeDtypeStruct((Q4, 4 * H), jnp.float32),
)

_msg32 = pl.pallas_call(
    _msg32_body,
    grid=(GRID_M,),
    in_specs=[
        pl.BlockSpec((BE4, 128), lambda i: (i, 0)),
        *_ea_specs(),
        pl.BlockSpec((512, KW), lambda i: (0, 0)),
        pl.BlockSpec((D_EDGE, D_EDGE * H), lambda i: (0, 0)),
    ],
    out_specs=pl.BlockSpec((BE4, 4 * H), lambda i: (i, 0)),
    out_shape=jax.ShapeDtypeStruct((Q4, 4 * H), jnp.float32),
)


def _update_body(x_ref, agg_ref, root_ref, bias_ref, out_ref):
    agg = agg_ref[pl.ds(0, N)] + agg_ref[pl.ds(NP, N)]
    h = (jnp.dot(x_ref[...], root_ref[...], preferred_element_type=jnp.float32)
         + agg + bias_ref[...])
    out_ref[...] = jnp.maximum(h, 0.0)


def _make_update(d_in):
    return pl.pallas_call(
        _update_body,
        out_shape=jax.ShapeDtypeStruct((N, H), jnp.float32),
    )


_update1 = _make_update(D_NODE)


def _final_body(h1_ref, agg_ref, root_ref, bias_ref, batch_ref,
                l1w_ref, l1b_ref, l2w_ref, l2b_ref, out_ref):
    agg = agg_ref[pl.ds(0, N)] + agg_ref[pl.ds(NP, N)]
    h2 = (jnp.dot(h1_ref[...], root_ref[...],
                  preferred_element_type=jnp.float32)
          + agg + bias_ref[...])
    gids = jnp.broadcast_to(batch_ref[...], (G, N))
    oht = (lax.broadcasted_iota(jnp.int32, (G, N), 0) == gids
           ).astype(jnp.float32)
    sums = jnp.dot(oht, h2, preferred_element_type=jnp.float32)
    cnt = jnp.sum(oht, axis=1, keepdims=True)
    pooled = sums / jnp.maximum(cnt, 1.0)
    z = jnp.maximum(
        jnp.dot(pooled, l1w_ref[...], preferred_element_type=jnp.float32)
        + l1b_ref[...], 0.0)
    out_ref[...] = jax.nn.sigmoid(
        jnp.dot(z, l2w_ref[...], preferred_element_type=jnp.float32)
        + l2b_ref[...])


_final = pl.pallas_call(
    _final_body,
    out_shape=jax.ShapeDtypeStruct((G, 1), jnp.float32),
)


def kernel(x, edge_index, edge_attr, batch,
           nn1_W, nn1_b, root1, bias1,
           nn2_W, nn2_b, root2, bias2,
           lin1_W, lin1_b, lin2_W, lin2_b):
    src = edge_index[0].astype(jnp.int32)
    dst = edge_index[1].astype(jnp.int32)
    pad = E_PAD - E

    def slots(a, m):
        return a.reshape(m, E_PAD // m).T.reshape(E_PAD)

    src_pad = jnp.concatenate([src, jnp.zeros((pad,), jnp.int32)])
    dst_pad = jnp.concatenate([dst, jnp.full((pad,), NP - 1, jnp.int32)])
    src_p1 = slots(src_pad, 2).reshape(NW, K_DMA, ROWS_DMA)
    src_p2 = slots(src_pad, 4).reshape(NW, K_DMA, ROWS_DMA)
    dst_p = slots(dst_pad, 4).reshape(NW, K_DMA, ROWS_DMA)
    eat = jnp.pad(edge_attr.T, ((0, 0), (0, pad)))

    def wcat(nnW, nnb, d_in):
        w = nnW.reshape(D_EDGE, d_in, H).transpose(1, 0, 2).reshape(d_in, -1)
        return jnp.concatenate([w, nnb.reshape(d_in, H)], axis=1)

    w1 = wcat(nn1_W, nn1_b, D_NODE)
    wall1 = jnp.concatenate(
        [jnp.pad(w1, ((64 * q, 64 * (1 - q)), (0, 0))) for q in (0, 1)],
        axis=0)
    w2 = wcat(nn2_W, nn2_b, H)
    wall2 = jnp.concatenate(
        [jnp.pad(w2, ((32 * q, 32 * (3 - q)), (0, 0))) for q in range(4)],
        axis=0)
    rmat = jnp.repeat(jnp.eye(D_EDGE, dtype=jnp.float32), H, axis=1)
    zeros_np = jnp.zeros((ROWS_PER_TILE, H), jnp.float32)
    batch2d = batch.astype(jnp.int32).reshape(1, N)

    xj = _make_gather(D_NODE)(x, src_p1).reshape(E_PAD // 2, 128)
    msg1 = _msg64(xj, xj, eat, eat, eat, eat, wall1, rmat)
    agg1 = _make_scatter()(msg1.reshape(E_PAD, H), dst_p, zeros_np)
    h1 = _update1(x, agg1, root1, bias1.reshape(1, H))

    hj = _make_gather(H)(h1, src_p2)
    msg2 = _msg32(hj.reshape(Q4, 128), eat, eat, eat, eat, wall2, rmat)
    agg2 = _make_scatter()(msg2.reshape(E_PAD, H), dst_p, zeros_np)

    return _final(h1, agg2, root2, bias2.reshape(1, H),
                  batch2d, lin1_W, lin1_b.reshape(1, H // 2),
                  lin2_W, lin2_b.reshape(1, 1))


# speedup vs baseline: 1.9709x; 1.1650x over previous
import functools

import jax
import jax.numpy as jnp
from jax import lax
from jax.experimental import pallas as pl
from jax.experimental.pallas import tpu as pltpu
from jax.experimental.pallas import tpu_sc as plsc

N = 10000
E = 160000
D_NODE = 64
D_EDGE = 16
H = 32
G = 64

NC = 2
NS = 16
NW = NC * NS
ROWS_DMA = 128
E_PAD = 163840
PER_W = E_PAD // NW
K_DMA = PER_W // ROWS_DMA
GROUP = 1024
N_GROUPS = PER_W // GROUP
DMA_PER_GROUP = GROUP // ROWS_DMA
NP = 10240
ROWS_PER_TILE = NP // NS

@functools.lru_cache(maxsize=None)
def _sc_mesh():
    return plsc.VectorSubcoreMesh(
        core_axis_name="c", subcore_axis_name="s",
        num_cores=NC, num_subcores=NS)


@functools.lru_cache(maxsize=None)
def _make_gather(d_in):
    group_rows = 32768 // d_in
    dpg = group_rows // ROWS_DMA
    ngr = PER_W // group_rows
    rpt = 632
    last = N - rpt * (NS - 1)

    def body(table_hbm, src_hbm, out_hbm, idx_v, rows_v, tab_sh, semg, semo):
        sid = lax.axis_index("s")
        wid = sid * NC + lax.axis_index("c")
        base = wid * PER_W
        pltpu.sync_copy(src_hbm.at[wid], idx_v)

        @pl.when(sid < NS - 1)
        def _():
            pltpu.sync_copy(table_hbm.at[pl.ds(sid * rpt, rpt)],
                            tab_sh.at[pl.ds(sid * rpt, rpt)])

        @pl.when(sid == NS - 1)
        def _():
            pltpu.sync_copy(table_hbm.at[pl.ds((NS - 1) * rpt, last)],
                            tab_sh.at[pl.ds((NS - 1) * rpt, last)])

        plsc.subcore_barrier()

        def fire(g, b):
            for j in range(dpg):
                pltpu.async_copy(
                    tab_sh.at[idx_v.at[g * dpg + j]],
                    rows_v.at[b, pl.ds(j * ROWS_DMA, ROWS_DMA)], semg)

        fire(0, 0)

        def group(g, carry):
            b = lax.rem(g, 2)
            nb = lax.rem(g + 1, 2)
            for j in range(dpg):
                pltpu.make_async_copy(
                    tab_sh.at[idx_v.at[j]],
                    rows_v.at[b, pl.ds(j * ROWS_DMA, ROWS_DMA)], semg).wait()

            @pl.when(g >= 1)
            def _():
                pltpu.make_async_copy(
                    rows_v.at[nb], out_hbm.at[pl.ds(base, group_rows)],
                    semo).wait()

            @pl.when(g + 1 < ngr)
            def _():
                fire(g + 1, nb)

            pltpu.async_copy(
                rows_v.at[b],
                out_hbm.at[pl.ds(base + g * group_rows, group_rows)], semo)
            return carry

        lax.fori_loop(0, ngr, group, 0)
        pltpu.make_async_copy(
            rows_v.at[lax.rem(ngr - 1, 2)],
            out_hbm.at[pl.ds(base, group_rows)], semo).wait()

    return pl.kernel(
        body,
        out_type=jax.ShapeDtypeStruct((E_PAD, d_in), jnp.float32),
        mesh=_sc_mesh(),
        compiler_params=pltpu.CompilerParams(use_tc_tiling_on_sc=False),
        scratch_types=[
            pltpu.VMEM((K_DMA, ROWS_DMA), jnp.int32),
            pltpu.VMEM((2, group_rows, d_in), jnp.float32),
            pltpu.VMEM_SHARED((N, d_in), jnp.float32),
            pltpu.SemaphoreType.DMA,
            pltpu.SemaphoreType.DMA,
        ],
    )


def _scatter_body(msg_hbm, dst_hbm, zeros_hbm, out_hbm,
                  idx_v, msg_v, acc_sh, seml, sems):
    cid = lax.axis_index("c")
    sid = lax.axis_index("s")
    wid = sid * NC + cid
    base = wid * PER_W
    row0 = sid * ROWS_PER_TILE
    pltpu.sync_copy(dst_hbm.at[wid], idx_v)
    pltpu.async_copy(msg_hbm.at[pl.ds(base, GROUP)], msg_v.at[0], seml)
    pltpu.sync_copy(zeros_hbm, acc_sh.at[pl.ds(row0, ROWS_PER_TILE)])
    plsc.subcore_barrier()

    def fire_scatters(g, b):
        for j in range(DMA_PER_GROUP):
            pltpu.async_copy(msg_v.at[b, pl.ds(j * ROWS_DMA, ROWS_DMA)],
                             acc_sh.at[idx_v.at[g * DMA_PER_GROUP + j]],
                             sems, add=True)

    def drain_scatters(b):
        for j in range(DMA_PER_GROUP):
            pltpu.make_async_copy(
                msg_v.at[b, pl.ds(j * ROWS_DMA, ROWS_DMA)],
                acc_sh.at[idx_v.at[j]], sems).wait()

    def group(g, carry):
        b = lax.rem(g, 2)
        nb = lax.rem(g + 1, 2)
        pltpu.make_async_copy(
            msg_hbm.at[pl.ds(base, GROUP)], msg_v.at[b], seml).wait()

        @pl.when(g >= 1)
        def _():
            drain_scatters(nb)

        @pl.when(g + 1 < N_GROUPS)
        def _():
            pltpu.async_copy(
                msg_hbm.at[pl.ds(base + (g + 1) * GROUP, GROUP)],
                msg_v.at[nb], seml)

        fire_scatters(g, b)
        return carry

    lax.fori_loop(0, N_GROUPS, group, 0)
    drain_scatters(lax.rem(N_GROUPS - 1, 2))
    plsc.subcore_barrier()
    pltpu.sync_copy(acc_sh.at[pl.ds(row0, ROWS_PER_TILE)],
                    out_hbm.at[pl.ds(cid * NP + row0, ROWS_PER_TILE)])


@functools.lru_cache(maxsize=None)
def _make_scatter():
    return pl.kernel(
        _scatter_body,
        out_type=jax.ShapeDtypeStruct((NC * NP, H), jnp.float32),
        mesh=_sc_mesh(),
        compiler_params=pltpu.CompilerParams(use_tc_tiling_on_sc=False),
        scratch_types=[
            pltpu.VMEM((K_DMA, ROWS_DMA), jnp.int32),
            pltpu.VMEM((2, GROUP, H), jnp.float32),
            pltpu.VMEM_SHARED((NP, H), jnp.float32),
            pltpu.SemaphoreType.DMA,
            pltpu.SemaphoreType.DMA,
        ],
    )


BE = 2048


KW = (D_EDGE + 1) * H
BE4 = 512
Q4 = E_PAD // 4
GRID_M = Q4 // BE4
NBLK = GRID_M


def _fold(c, y):
    t = c * y[:, :D_EDGE * H]
    a = t[:, :256] + t[:, 256:512]
    a = a[:, :128] + a[:, 128:]
    a = a[:, :64] + a[:, 64:]
    return a[:, :H] + a[:, H:] + y[:, D_EDGE * H:]


def _cexp(eat, r_ref):
    return jax.lax.dot_general(eat, r_ref[...], (((0,), (0,)), ((), ())),
                               preferred_element_type=jnp.float32)


def _msg64_body(xa_ref, xb_ref, ea0_ref, ea1_ref, ea2_ref, ea3_ref,
                w_ref, r_ref, out_ref):
    blka = xa_ref[...]
    blkb = xb_ref[...]
    wl = w_ref[pl.ds(0, 128), :]
    wr = w_ref[pl.ds(128, 128), :]
    m0 = _fold(_cexp(ea0_ref[...], r_ref),
               jnp.dot(blka, wl, preferred_element_type=jnp.float32))
    m1 = _fold(_cexp(ea1_ref[...], r_ref),
               jnp.dot(blkb, wl, preferred_element_type=jnp.float32))
    m2 = _fold(_cexp(ea2_ref[...], r_ref),
               jnp.dot(blka, wr, preferred_element_type=jnp.float32))
    m3 = _fold(_cexp(ea3_ref[...], r_ref),
               jnp.dot(blkb, wr, preferred_element_type=jnp.float32))
    out_ref[...] = jnp.concatenate([m0, m1, m2, m3], axis=1)


def _msg32_body(xj_ref, ea0_ref, ea1_ref, ea2_ref, ea3_ref,
                w_ref, r_ref, out_ref):
    blk = xj_ref[...]
    ea_refs = (ea0_ref, ea1_ref, ea2_ref, ea3_ref)
    msgs = []
    for q in range(4):
        y = jnp.dot(blk, w_ref[pl.ds(q * 128, 128), :],
                    preferred_element_type=jnp.float32)
        msgs.append(_fold(_cexp(ea_refs[q][...], r_ref), y))
    out_ref[...] = jnp.concatenate(msgs, axis=1)


def _ea_specs():
    return [pl.BlockSpec((D_EDGE, BE4), lambda i, q=q: (0, q * NBLK + i))
            for q in range(4)]


_msg64 = pl.pallas_call(
    _msg64_body,
    grid=(GRID_M,),
    in_specs=[
        pl.BlockSpec((BE4, 128), lambda i: (i, 0)),
        pl.BlockSpec((BE4, 128), lambda i: (GRID_M + i, 0)),
        *_ea_specs(),
        pl.BlockSpec((256, KW), lambda i: (0, 0)),
        pl.BlockSpec((D_EDGE, D_EDGE * H), lambda i: (0, 0)),
    ],
    out_specs=pl.BlockSpec((BE4, 4 * H), lambda i: (i, 0)),
    out_shape=jax.ShapeDtypeStruct((Q4, 4 * H), jnp.float32),
)

_msg32 = pl.pallas_call(
    _msg32_body,
    grid=(GRID_M,),
    in_specs=[
        pl.BlockSpec((BE4, 128), lambda i: (i, 0)),
        *_ea_specs(),
        pl.BlockSpec((512, KW), lambda i: (0, 0)),
        pl.BlockSpec((D_EDGE, D_EDGE * H), lambda i: (0, 0)),
    ],
    out_specs=pl.BlockSpec((BE4, 4 * H), lambda i: (i, 0)),
    out_shape=jax.ShapeDtypeStruct((Q4, 4 * H), jnp.float32),
)


def _update_body(x_ref, agg_ref, root_ref, bias_ref, out_ref):
    agg = agg_ref[pl.ds(0, N)] + agg_ref[pl.ds(NP, N)]
    h = (jnp.dot(x_ref[...], root_ref[...], preferred_element_type=jnp.float32)
         + agg + bias_ref[...])
    out_ref[...] = jnp.maximum(h, 0.0)


def _make_update(d_in):
    return pl.pallas_call(
        _update_body,
        out_shape=jax.ShapeDtypeStruct((N, H), jnp.float32),
    )


_update1 = _make_update(D_NODE)


def _final_body(h1_ref, agg_ref, root_ref, bias_ref, batch_ref,
                l1w_ref, l1b_ref, l2w_ref, l2b_ref, out_ref):
    agg = agg_ref[pl.ds(0, N)] + agg_ref[pl.ds(NP, N)]
    h2 = (jnp.dot(h1_ref[...], root_ref[...],
                  preferred_element_type=jnp.float32)
          + agg + bias_ref[...])
    gids = jnp.broadcast_to(batch_ref[...], (G, N))
    oht = (lax.broadcasted_iota(jnp.int32, (G, N), 0) == gids
           ).astype(jnp.float32)
    sums = jnp.dot(oht, h2, preferred_element_type=jnp.float32)
    cnt = jnp.sum(oht, axis=1, keepdims=True)
    pooled = sums / jnp.maximum(cnt, 1.0)
    z = jnp.maximum(
        jnp.dot(pooled, l1w_ref[...], preferred_element_type=jnp.float32)
        + l1b_ref[...], 0.0)
    out_ref[...] = jax.nn.sigmoid(
        jnp.dot(z, l2w_ref[...], preferred_element_type=jnp.float32)
        + l2b_ref[...])


_final = pl.pallas_call(
    _final_body,
    out_shape=jax.ShapeDtypeStruct((G, 1), jnp.float32),
)


def kernel(x, edge_index, edge_attr, batch,
           nn1_W, nn1_b, root1, bias1,
           nn2_W, nn2_b, root2, bias2,
           lin1_W, lin1_b, lin2_W, lin2_b):
    src = edge_index[0].astype(jnp.int32)
    dst = edge_index[1].astype(jnp.int32)
    pad = E_PAD - E

    def slots(a, m):
        ep = E_PAD // m
        parts = [a[q * ep:(q + 1) * ep].reshape(-1, 128 // m)
                 for q in range(m)]
        return jnp.stack(parts, axis=2).reshape(E_PAD)

    src_pad = jnp.concatenate([src, jnp.zeros((pad,), jnp.int32)])
    dst_pad = jnp.concatenate([dst, jnp.full((pad,), NP - 1, jnp.int32)])
    src_p1 = slots(src_pad, 2).reshape(NW, K_DMA, ROWS_DMA)
    src_p2 = slots(src_pad, 4).reshape(NW, K_DMA, ROWS_DMA)
    dst_p = slots(dst_pad, 4).reshape(NW, K_DMA, ROWS_DMA)
    eat = jnp.pad(edge_attr.T, ((0, 0), (0, pad)))

    def wcat(nnW, nnb, d_in):
        w = nnW.reshape(D_EDGE, d_in, H).transpose(1, 0, 2).reshape(d_in, -1)
        return jnp.concatenate([w, nnb.reshape(d_in, H)], axis=1)

    w1 = wcat(nn1_W, nn1_b, D_NODE)
    wall1 = jnp.concatenate(
        [jnp.pad(w1, ((64 * q, 64 * (1 - q)), (0, 0))) for q in (0, 1)],
        axis=0)
    w2 = wcat(nn2_W, nn2_b, H)
    wall2 = jnp.concatenate(
        [jnp.pad(w2, ((32 * q, 32 * (3 - q)), (0, 0))) for q in range(4)],
        axis=0)
    rmat = jnp.repeat(jnp.eye(D_EDGE, dtype=jnp.float32), H, axis=1)
    zeros_np = jnp.zeros((ROWS_PER_TILE, H), jnp.float32)
    batch2d = batch.astype(jnp.int32).reshape(1, N)

    xj = _make_gather(D_NODE)(x, src_p1).reshape(E_PAD // 2, 128)
    msg1 = _msg64(xj, xj, eat, eat, eat, eat, wall1, rmat)
    agg1 = _make_scatter()(msg1.reshape(E_PAD, H), dst_p, zeros_np)
    h1 = _update1(x, agg1, root1, bias1.reshape(1, H))

    hj = _make_gather(H)(h1, src_p2)
    msg2 = _msg32(hj.reshape(Q4, 128), eat, eat, eat, eat, wall2, rmat)
    agg2 = _make_scatter()(msg2.reshape(E_PAD, H), dst_p, zeros_np)

    return _final(h1, agg2, root2, bias2.reshape(1, H),
                  batch2d, lin1_W, lin1_b.reshape(1, H // 2),
                  lin2_W, lin2_b.reshape(1, 1))
